# Initial kernel scaffold; baseline (speedup 1.0000x reference)
import jax
import jax.numpy as jnp
from jax.experimental import pallas as pl


def kernel(x, edge_index, W1l, b1, W1r, W2l, b2, W2r, W3l, b3, W3r):
    raise NotImplementedError("write your pallas kernel here")


# speedup vs baseline: 12.9379x; 12.9379x over previous
import functools

import jax
import jax.numpy as jnp
from jax import lax
from jax.experimental import pallas as pl
from jax.experimental.pallas import tpu as pltpu
from jax.experimental.pallas import tpu_sc as plsc

N = 10000
E = 320000
NC = 2
NS = 16
NW = NC * NS
CW = 80
NCH = E // NW // CW
RPT = N // NS

D1P = 144


def _make_agg(D):
  mesh = plsc.VectorSubcoreMesh(core_axis_name="c", subcore_axis_name="s")

  @functools.partial(
      pl.kernel,
      mesh=mesh,
      compiler_params=pltpu.CompilerParams(use_tc_tiling_on_sc=False),
      out_type=jax.ShapeDtypeStruct((NC, N, D), jnp.float32),
      scratch_types=[
          pltpu.VMEM((NCH, CW), jnp.int32),
          pltpu.VMEM((NCH, CW), jnp.int32),
          pltpu.VMEM((CW, D), jnp.float32),
          pltpu.VMEM_SHARED((N, D), jnp.float32),
          pltpu.SemaphoreType.DMA,
      ],
  )
  def agg(table, srcr, dstr, zeros, out, src_v, dst_v, rows_v, acc, sem):
    c = lax.axis_index("c")
    s = lax.axis_index("s")
    wid = s * NC + c

    pltpu.sync_copy(zeros.at[pl.ds(s * RPT, RPT)], acc.at[pl.ds(s * RPT, RPT)])
    pltpu.sync_copy(srcr.at[wid], src_v)
    pltpu.sync_copy(dstr.at[wid], dst_v)
    plsc.subcore_barrier()

    def body(j, carry):
      pltpu.async_copy(table.at[src_v.at[j]], rows_v, sem).wait()
      pltpu.sync_copy(rows_v, acc.at[dst_v.at[j]], add=True)
      return carry

    lax.fori_loop(0, NCH, body, 0)

    plsc.subcore_barrier()
    pltpu.sync_copy(acc.at[pl.ds(s * RPT, RPT)],
                    out.at[c].at[pl.ds(s * RPT, RPT)])

  return agg


_agg144 = _make_agg(D1P)
_agg128 = _make_agg(128)
_agg64 = _make_agg(64)


NB = 1000
GRID = N // NB


def _layer1_body(acc_ref, x_ref, wl_ref, wr_ref, b_ref, h_ref, rdeg_ref):
  acc = acc_ref[0] + acc_ref[1]
  agg = acc[:, :128]
  deg = acc[:, 128:129]
  rdeg = 1.0 / jnp.maximum(deg, 1.0)
  h = jnp.dot(agg * rdeg, wl_ref[...], preferred_element_type=jnp.float32)
  h = h + jnp.dot(x_ref[...], wr_ref[...], preferred_element_type=jnp.float32)
  h = h + b_ref[...][None, :]
  h_ref[...] = jnp.maximum(h, 0.0)
  rdeg_ref[...] = jnp.broadcast_to(rdeg, (NB, 8))


def _layer1(acc1, x, W1l, W1r, b1):
  return pl.pallas_call(
      _layer1_body,
      grid=(GRID,),
      in_specs=[
          pl.BlockSpec((NC, NB, D1P), lambda i: (0, i, 0)),
          pl.BlockSpec((NB, 128), lambda i: (i, 0)),
          pl.BlockSpec((128, 128), lambda i: (0, 0)),
          pl.BlockSpec((128, 128), lambda i: (0, 0)),
          pl.BlockSpec((128,), lambda i: (0,)),
      ],
      out_specs=[
          pl.BlockSpec((NB, 128), lambda i: (i, 0)),
          pl.BlockSpec((NB, 8), lambda i: (i, 0)),
      ],
      out_shape=[
          jax.ShapeDtypeStruct((N, 128), jnp.float32),
          jax.ShapeDtypeStruct((N, 8), jnp.float32),
      ],
  )(acc1, x, W1l, W1r, b1)


def _layer23_body(acc_ref, rdeg_ref, h1_ref, w2l_ref, w2r_ref, b2_ref,
                  w3l_ref, w3r_ref, p3_ref, q3_ref):
  mean = (acc_ref[0] + acc_ref[1]) * rdeg_ref[...][:, :1]
  h2 = jnp.dot(mean, w2l_ref[...], preferred_element_type=jnp.float32)
  h2 = h2 + jnp.dot(h1_ref[...], w2r_ref[...],
                    preferred_element_type=jnp.float32)
  h2 = jnp.maximum(h2 + b2_ref[...][None, :], 0.0)
  p3_ref[...] = jnp.dot(h2, w3l_ref[...], preferred_element_type=jnp.float32)
  q3_ref[...] = jnp.dot(h2, w3r_ref[...], preferred_element_type=jnp.float32)


def _layer23(acc2, rdeg, h1, W2l, W2r, b2, W3l, W3r):
  return pl.pallas_call(
      _layer23_body,
      grid=(GRID,),
      in_specs=[
          pl.BlockSpec((NC, NB, 128), lambda i: (0, i, 0)),
          pl.BlockSpec((NB, 8), lambda i: (i, 0)),
          pl.BlockSpec((NB, 128), lambda i: (i, 0)),
          pl.BlockSpec((128, 1024), lambda i: (0, 0)),
          pl.BlockSpec((128, 1024), lambda i: (0, 0)),
          pl.BlockSpec((1024,), lambda i: (0,)),
          pl.BlockSpec((1024, 64), lambda i: (0, 0)),
          pl.BlockSpec((1024, 64), lambda i: (0, 0)),
      ],
      out_specs=[
          pl.BlockSpec((NB, 64), lambda i: (i, 0)),
          pl.BlockSpec((NB, 64), lambda i: (i, 0)),
      ],
      out_shape=[
          jax.ShapeDtypeStruct((N, 64), jnp.float32),
          jax.ShapeDtypeStruct((N, 64), jnp.float32),
      ],
  )(acc2, rdeg, h1, W2l, W2r, b2, W3l, W3r)


def _final_body(acc_ref, rdeg_ref, q3_ref, b3_ref, out_ref):
  h3 = (acc_ref[0] + acc_ref[1]) * rdeg_ref[...][:, :1]
  h3 = h3 + q3_ref[...] + b3_ref[...][None, :]
  m = jnp.max(h3, axis=1, keepdims=True)
  lse = jnp.log(jnp.sum(jnp.exp(h3 - m), axis=1, keepdims=True)) + m
  out_ref[...] = h3 - lse


def _final(acc3, rdeg, q3, b3):
  return pl.pallas_call(
      _final_body,
      grid=(GRID,),
      in_specs=[
          pl.BlockSpec((NC, NB, 64), lambda i: (0, i, 0)),
          pl.BlockSpec((NB, 8), lambda i: (i, 0)),
          pl.BlockSpec((NB, 64), lambda i: (i, 0)),
          pl.BlockSpec((64,), lambda i: (0,)),
      ],
      out_specs=pl.BlockSpec((NB, 64), lambda i: (i, 0)),
      out_shape=jax.ShapeDtypeStruct((N, 64), jnp.float32),
  )(acc3, rdeg, q3, b3)


def kernel(x, edge_index, W1l, b1, W1r, W2l, b2, W2r, W3l, b3, W3r):
  src = edge_index[0].reshape(NW, NCH, CW)
  dst = edge_index[1].reshape(NW, NCH, CW)

  pad = jnp.concatenate(
      [jnp.ones((N, 1), jnp.float32), jnp.zeros((N, D1P - 129), jnp.float32)],
      axis=1)
  xp = jnp.concatenate([x, pad], axis=1)

  z144 = jnp.zeros((N, D1P), jnp.float32)
  z128 = jnp.zeros((N, 128), jnp.float32)
  z64 = jnp.zeros((N, 64), jnp.float32)

  acc1 = _agg144(xp, src, dst, z144)
  h1, rdeg = _layer1(acc1, x, W1l, W1r, b1)
  acc2 = _agg128(h1, src, dst, z128)
  p3, q3 = _layer23(acc2, rdeg, h1, W2l, W2r, b2, W3l, W3r)
  acc3 = _agg64(p3, src, dst, z64)
  return _final(acc3, rdeg, q3, b3)
